# baseline (speedup 1.0000x reference)
import functools

import numpy as np
import jax
import jax.numpy as jnp
from jax import lax
from jax.experimental import pallas as pl
from jax.experimental.pallas import tpu as pltpu
from jax.experimental.pallas import tpu_sc as plsc

H = 0.001
Q = 3
N_COLS = 1024
N_XS = 16384

NC, NS, L = 2, 16, 16
NW = NC * NS
RPW = N_XS // NW
CR = 32
NCHUNK = RPW // CR
GPC = CR // L

_XE_OFF = [float(np.float32(H) * np.float32(Q - j)) for j in range(Q + 1)]

_mesh = plsc.VectorSubcoreMesh(
    core_axis_name="c", subcore_axis_name="s", num_cores=NC, num_subcores=NS
)


@functools.partial(
    pl.kernel,
    out_type=jax.ShapeDtypeStruct((N_XS * N_COLS,), jnp.float32),
    mesh=_mesh,
    scratch_types=[
        pltpu.VMEM((RPW,), jnp.float32),
        pltpu.VMEM((2 * L,), jnp.float32),
        pltpu.VMEM((CR * N_COLS,), jnp.float32),
        pltpu.VMEM((CR * N_COLS,), jnp.float32),
        pltpu.SemaphoreType.DMA,
        pltpu.SemaphoreType.DMA,
    ],
    compiler_params=pltpu.CompilerParams(needs_layout_passes=False),
)
def _sc_band(xs_hbm, b_hbm, zeros_hbm, out_hbm, xs_v, b_v, buf0, buf1, sem0, sem1):
    wid = lax.axis_index("s") * NC + lax.axis_index("c")
    row0 = wid * RPW
    pltpu.sync_copy(xs_hbm.at[pl.ds(row0, RPW)], xs_v)
    pltpu.sync_copy(b_hbm, b_v)
    pltpu.sync_copy(zeros_hbm, buf0)
    pltpu.sync_copy(zeros_hbm, buf1)

    lanes = lax.iota(jnp.int32, L)
    coefs = [
        [plsc.load_gather(b_v, [jnp.full((L,), L + 4 * j + p, jnp.int32)])
         for p in range(Q + 1)]
        for j in range(Q + 1)
    ]
    zero16 = jnp.zeros((L,), jnp.float32)

    def group_first_i(c, g):
        x = xs_v[pl.ds(c * CR + g * L, L)]
        fi = (x / H).astype(jnp.int32)
        return x, fi

    def fill(buf, c):
        for g in range(GPC):
            x, fi = group_first_i(c, g)
            xm = x - fi.astype(jnp.float32) * H
            base = (g * L + lanes) * N_COLS + fi
            for j in range(Q + 1):
                xe = xm + _XE_OFF[j]
                cj = coefs[j]
                v = ((cj[3] * xe + cj[2]) * xe + cj[1]) * xe + cj[0]
                plsc.store_scatter(buf, [base + j], v)

    def unscatter(buf, c):
        for g in range(GPC):
            _, fi = group_first_i(c, g)
            base = (g * L + lanes) * N_COLS + fi
            for j in range(Q + 1):
                plsc.store_scatter(buf, [base + j], zero16)

    bufs = (buf0, buf1)
    sems = (sem0, sem1)
    copies = [None, None]
    for c in range(NCHUNK):
        s = c & 1
        if copies[s] is not None:
            copies[s].wait()
            unscatter(bufs[s], c - 2)
        fill(bufs[s], c)
        copies[s] = pltpu.async_copy(
            bufs[s],
            out_hbm.at[pl.ds((row0 + c * CR) * N_COLS, CR * N_COLS)],
            sems[s],
        )
    copies[0].wait()
    copies[1].wait()


@jax.jit
def kernel(xs, B):
    zeros = jnp.zeros((CR * N_COLS,), jnp.float32)
    b2 = jnp.concatenate([B.reshape(-1), B.reshape(-1)])
    out_flat = _sc_band(xs, b2, zeros)
    return out_flat.reshape(N_XS, N_COLS)


# speedup vs baseline: 2.2314x; 2.2314x over previous
import functools

import numpy as np
import jax
import jax.numpy as jnp
from jax import lax
from jax.experimental import pallas as pl
from jax.experimental.pallas import tpu as pltpu
from jax.experimental.pallas import tpu_sc as plsc

H = 0.001
Q = 3
N_COLS = 1024
N_XS = 16384

NC, NS, L = 2, 16, 16
NW = NC * NS
RPW = N_XS // NW
CR = 32
NCHUNK = RPW // CR
GPC = CR // L

_XE_OFF = [float(np.float32(H) * np.float32(Q - j)) for j in range(Q + 1)]

_mesh = plsc.VectorSubcoreMesh(
    core_axis_name="c", subcore_axis_name="s", num_cores=NC, num_subcores=NS
)


@functools.partial(
    pl.kernel,
    out_type=jax.ShapeDtypeStruct((N_XS, N_COLS), jnp.float32),
    mesh=_mesh,
    scratch_types=[
        pltpu.VMEM((RPW,), jnp.float32),
        pltpu.VMEM((2 * L,), jnp.float32),
        pltpu.VMEM((CR, N_COLS), jnp.float32),
        pltpu.VMEM((CR, N_COLS), jnp.float32),
        pltpu.SemaphoreType.DMA,
        pltpu.SemaphoreType.DMA,
    ],
    compiler_params=pltpu.CompilerParams(needs_layout_passes=False),
)
def _sc_band(xs_hbm, b_hbm, zeros_hbm, out_hbm, xs_v, b_v, buf0, buf1, sem0, sem1):
    wid = lax.axis_index("s") * NC + lax.axis_index("c")
    row0 = wid * RPW
    pltpu.sync_copy(xs_hbm.at[pl.ds(row0, RPW)], xs_v)
    pltpu.sync_copy(b_hbm, b_v)
    pltpu.sync_copy(zeros_hbm, buf0)
    pltpu.sync_copy(zeros_hbm, buf1)

    lanes = lax.iota(jnp.int32, L)
    coefs = [
        [plsc.load_gather(b_v, [jnp.full((L,), L + 4 * j + p, jnp.int32)])
         for p in range(Q + 1)]
        for j in range(Q + 1)
    ]
    zero16 = jnp.zeros((L,), jnp.float32)

    def group_first_i(c, g):
        x = xs_v[pl.ds(c * CR + g * L, L)]
        fi = (x / H).astype(jnp.int32)
        return x, fi

    def fill(buf, c):
        for g in range(GPC):
            x, fi = group_first_i(c, g)
            xm = x - fi.astype(jnp.float32) * H
            rows = g * L + lanes
            for j in range(Q + 1):
                xe = xm + _XE_OFF[j]
                cj = coefs[j]
                v = ((cj[3] * xe + cj[2]) * xe + cj[1]) * xe + cj[0]
                plsc.store_scatter(buf, [rows, fi + j], v)

    def unscatter(buf, c):
        for g in range(GPC):
            _, fi = group_first_i(c, g)
            rows = g * L + lanes
            for j in range(Q + 1):
                plsc.store_scatter(buf, [rows, fi + j], zero16)

    bufs = (buf0, buf1)
    sems = (sem0, sem1)
    copies = [None, None]
    for c in range(NCHUNK):
        s = c & 1
        if copies[s] is not None:
            copies[s].wait()
            unscatter(bufs[s], c - 2)
        fill(bufs[s], c)
        copies[s] = pltpu.async_copy(
            bufs[s],
            out_hbm.at[pl.ds(row0 + c * CR, CR)],
            sems[s],
        )
    copies[0].wait()
    copies[1].wait()


@jax.jit
def kernel(xs, B):
    zeros = jnp.zeros((CR, N_COLS), jnp.float32)
    b2 = jnp.concatenate([B.reshape(-1), B.reshape(-1)])
    return _sc_band(xs, b2, zeros)


# speedup vs baseline: 2.2585x; 1.0121x over previous
import functools

import numpy as np
import jax
import jax.numpy as jnp
from jax import lax
from jax.experimental import pallas as pl
from jax.experimental.pallas import tpu as pltpu
from jax.experimental.pallas import tpu_sc as plsc

H = 0.001
Q = 3
N_COLS = 1024
N_XS = 16384

NC, NS, L = 2, 16, 16
NW = NC * NS
RPW = N_XS // NW
CR = 32
NCHUNK = RPW // CR
GPC = CR // L

_XE_OFF = [float(np.float32(H) * np.float32(Q - j)) for j in range(Q + 1)]

_mesh = plsc.VectorSubcoreMesh(
    core_axis_name="c", subcore_axis_name="s", num_cores=NC, num_subcores=NS
)


@functools.partial(
    pl.kernel,
    out_type=jax.ShapeDtypeStruct((N_XS, N_COLS), jnp.float32),
    mesh=_mesh,
    scratch_types=[
        pltpu.VMEM((RPW,), jnp.float32),
        pltpu.VMEM((2 * L,), jnp.float32),
        pltpu.VMEM((CR, N_COLS), jnp.float32),
        pltpu.VMEM((CR, N_COLS), jnp.float32),
        pltpu.SemaphoreType.DMA,
        pltpu.SemaphoreType.DMA,
    ],
    compiler_params=pltpu.CompilerParams(needs_layout_passes=False),
)
def _sc_band(xs_hbm, b_hbm, out_hbm, xs_v, b_v, buf0, buf1, sem0, sem1):
    wid = lax.axis_index("s") * NC + lax.axis_index("c")
    row0 = wid * RPW
    pltpu.sync_copy(xs_hbm.at[pl.ds(row0, RPW)], xs_v)
    pltpu.sync_copy(b_hbm, b_v)

    lanes = lax.iota(jnp.int32, L)
    coefs = [
        [plsc.load_gather(b_v, [jnp.full((L,), L + 4 * j + p, jnp.int32)])
         for p in range(Q + 1)]
        for j in range(Q + 1)
    ]
    zero16 = jnp.zeros((L,), jnp.float32)
    bufs = (buf0, buf1)
    sems = (sem0, sem1)

    def zero_buf(buf):
        @pl.loop(0, CR)
        def _(r):
            @pl.loop(0, N_COLS // L)
            def _(ci):
                buf[r, pl.ds(ci * L, L)] = zero16

    def group_first_i(c, g):
        off = pl.multiple_of(c * CR + g * L, L)
        x = xs_v[pl.ds(off, L)]
        fi = (x / H).astype(jnp.int32)
        return x, fi

    def fill(buf, c):
        for g in range(GPC):
            x, fi = group_first_i(c, g)
            xm = x - fi.astype(jnp.float32) * H
            rows = g * L + lanes
            for j in range(Q + 1):
                xe = xm + _XE_OFF[j]
                cj = coefs[j]
                v = ((cj[3] * xe + cj[2]) * xe + cj[1]) * xe + cj[0]
                plsc.store_scatter(buf, [rows, fi + j], v)

    def unscatter(buf, c):
        for g in range(GPC):
            _, fi = group_first_i(c, g)
            rows = g * L + lanes
            for j in range(Q + 1):
                plsc.store_scatter(buf, [rows, fi + j], zero16)

    def start_dma(s, c):
        return pltpu.async_copy(
            bufs[s], out_hbm.at[pl.ds(row0 + c * CR, CR)], sems[s]
        )

    zero_buf(buf0)
    fill(buf0, 0)
    start_dma(0, 0)
    zero_buf(buf1)
    fill(buf1, 1)
    start_dma(1, 1)

    @pl.loop(1, NCHUNK // 2)
    def _(cc):
        for s in range(2):
            c = cc * 2 + s
            pltpu.make_async_copy(
                bufs[s], out_hbm.at[pl.ds(row0 + (c - 2) * CR, CR)], sems[s]
            ).wait()
            unscatter(bufs[s], c - 2)
            fill(bufs[s], c)
            start_dma(s, c)

    for s in range(2):
        pltpu.make_async_copy(
            bufs[s], out_hbm.at[pl.ds(row0 + (NCHUNK - 2 + s) * CR, CR)], sems[s]
        ).wait()


@jax.jit
def kernel(xs, B):
    b2 = jnp.concatenate([B.reshape(-1), B.reshape(-1)])
    return _sc_band(xs, b2)


# speedup vs baseline: 2.2663x; 1.0034x over previous
import functools

import numpy as np
import jax
import jax.numpy as jnp
from jax import lax
from jax.experimental import pallas as pl
from jax.experimental.pallas import tpu as pltpu
from jax.experimental.pallas import tpu_sc as plsc

H = 0.001
Q = 3
N_COLS = 1024
N_XS = 16384

NC, NS, L = 2, 16, 16
NW = NC * NS
RPW = N_XS // NW
CR = 32
NCHUNK = RPW // CR
GPC = CR // L

_XE_OFF = [float(np.float32(H) * np.float32(Q - j)) for j in range(Q + 1)]

_mesh = plsc.VectorSubcoreMesh(
    core_axis_name="c", subcore_axis_name="s", num_cores=NC, num_subcores=NS
)


@functools.partial(
    pl.kernel,
    out_type=jax.ShapeDtypeStruct((N_XS, N_COLS), jnp.float32),
    mesh=_mesh,
    scratch_types=[
        pltpu.VMEM((RPW,), jnp.float32),
        pltpu.VMEM((2 * L,), jnp.float32),
        pltpu.VMEM((CR, N_COLS), jnp.float32),
        pltpu.VMEM((CR, N_COLS), jnp.float32),
        pltpu.SemaphoreType.DMA,
        pltpu.SemaphoreType.DMA,
    ],
    compiler_params=pltpu.CompilerParams(needs_layout_passes=False),
)
def _sc_band(xs_hbm, b_hbm, out_hbm, xs_v, b_v, buf0, buf1, sem0, sem1):
    wid = lax.axis_index("s") * NC + lax.axis_index("c")
    row0 = wid * RPW
    pltpu.sync_copy(xs_hbm.at[pl.ds(row0, RPW)], xs_v)
    pltpu.sync_copy(b_hbm, b_v.at[pl.ds(0, L)])
    pltpu.sync_copy(b_hbm, b_v.at[pl.ds(L, L)])

    lanes = lax.iota(jnp.int32, L)
    coefs = [
        [plsc.load_gather(b_v, [jnp.full((L,), L + 4 * j + p, jnp.int32)])
         for p in range(Q + 1)]
        for j in range(Q + 1)
    ]
    zero16 = jnp.zeros((L,), jnp.float32)
    bufs = (buf0, buf1)
    sems = (sem0, sem1)

    def zero_buf(buf):
        @pl.loop(0, CR)
        def _(r):
            @pl.loop(0, N_COLS // L)
            def _(ci):
                buf[r, pl.ds(ci * L, L)] = zero16

    def group_first_i(c, g):
        off = pl.multiple_of(c * CR + g * L, L)
        x = xs_v[pl.ds(off, L)]
        fi = (x / H).astype(jnp.int32)
        return x, fi

    def fill(buf, c):
        for g in range(GPC):
            x, fi = group_first_i(c, g)
            xm = x - fi.astype(jnp.float32) * H
            rows = g * L + lanes
            for j in range(Q + 1):
                xe = xm + _XE_OFF[j]
                cj = coefs[j]
                v = ((cj[3] * xe + cj[2]) * xe + cj[1]) * xe + cj[0]
                plsc.store_scatter(buf, [rows, fi + j], v)

    def unscatter(buf, c):
        for g in range(GPC):
            _, fi = group_first_i(c, g)
            rows = g * L + lanes
            for j in range(Q + 1):
                plsc.store_scatter(buf, [rows, fi + j], zero16)

    def start_dma(s, c):
        return pltpu.async_copy(
            bufs[s], out_hbm.at[pl.ds(row0 + c * CR, CR)], sems[s]
        )

    zero_buf(buf0)
    fill(buf0, 0)
    start_dma(0, 0)
    zero_buf(buf1)
    fill(buf1, 1)
    start_dma(1, 1)

    @pl.loop(1, NCHUNK // 2)
    def _(cc):
        for s in range(2):
            c = cc * 2 + s
            pltpu.make_async_copy(
                bufs[s], out_hbm.at[pl.ds(row0 + (c - 2) * CR, CR)], sems[s]
            ).wait()
            unscatter(bufs[s], c - 2)
            fill(bufs[s], c)
            start_dma(s, c)

    for s in range(2):
        pltpu.make_async_copy(
            bufs[s], out_hbm.at[pl.ds(row0 + (NCHUNK - 2 + s) * CR, CR)], sems[s]
        ).wait()


@jax.jit
def kernel(xs, B):
    return _sc_band(xs, B.reshape(-1))
